# baseline (speedup 1.0000x reference)
import functools

import jax
import jax.numpy as jnp
from jax import lax
from jax.experimental import pallas as pl
from jax.experimental.pallas import tpu as pltpu
from jax.experimental.pallas import tpu_sc as plsc

ROWS = 128
COLS = 100000
LANES = 16

_info = plsc.get_sparse_core_info()
_NC, _NS = _info.num_cores, _info.num_subcores
NWORKERS = _NC * _NS
ROWS_PER_W = ROWS // NWORKERS
CHUNK = 20000
CHUNKS_PER_ROW = COLS // CHUNK
VECS_PER_CHUNK = CHUNK // LANES

_NEG_INF = float("-inf")
_BIG_I32 = 0x7FFFFFFF


@functools.partial(
    pl.kernel,
    out_type=jax.ShapeDtypeStruct((NWORKERS, LANES), jnp.int32),
    mesh=plsc.VectorSubcoreMesh(core_axis_name="c", subcore_axis_name="s"),
    compiler_params=pltpu.CompilerParams(needs_layout_passes=False),
    scratch_types=[
        pltpu.VMEM((CHUNK,), jnp.float32),
        pltpu.VMEM((LANES,), jnp.int32),
    ],
)
def _sc_argmax(x_hbm, out_hbm, buf, out_v):
    wid = lax.axis_index("s") * _NC + lax.axis_index("c")
    iota16 = lax.iota(jnp.int32, LANES)
    acc = jnp.zeros((LANES,), jnp.int32)

    for r in range(ROWS_PER_W):
        row = wid * ROWS_PER_W + r
        row_base = row * COLS
        m = jnp.full((LANES,), _NEG_INF, jnp.float32)
        a = jnp.zeros((LANES,), jnp.int32)
        for c in range(CHUNKS_PER_ROW):
            pltpu.sync_copy(x_hbm.at[pl.ds(row_base + c * CHUNK, CHUNK)], buf)
            chunk_base = c * CHUNK

            def body(i, carry, chunk_base=chunk_base):
                mm, aa = carry
                v = buf[pl.ds(i * LANES, LANES)]
                idxv = iota16 + (chunk_base + i * LANES)
                gt = v > mm
                return jnp.where(gt, v, mm), jnp.where(gt, idxv, aa)

            m, a = lax.fori_loop(0, VECS_PER_CHUNK, body, (m, a))

        best = jnp.max(m)
        cand = jnp.where(m == best, a, _BIG_I32)
        acc = jnp.where(iota16 == r, jnp.min(cand), acc)

    out_v[...] = acc
    pltpu.sync_copy(out_v, out_hbm.at[wid])


def kernel(m_logits):
    flat = m_logits.reshape(-1)
    out = _sc_argmax(flat)
    return out[:, :ROWS_PER_W].reshape(ROWS, 1)


# speedup vs baseline: 1.6759x; 1.6759x over previous
import functools

import jax
import jax.numpy as jnp
from jax import lax
from jax.experimental import pallas as pl
from jax.experimental.pallas import tpu as pltpu
from jax.experimental.pallas import tpu_sc as plsc

ROWS = 128
COLS = 100000
LANES = 16

_info = plsc.get_sparse_core_info()
_NC, _NS = _info.num_cores, _info.num_subcores
NWORKERS = _NC * _NS
ROWS_PER_W = ROWS // NWORKERS
CHUNK = 20000
CHUNKS_PER_ROW = COLS // CHUNK
NCHUNKS = ROWS_PER_W * CHUNKS_PER_ROW
UNROLL = 10
ITERS = CHUNK // (LANES * UNROLL)

_NEG_INF = float("-inf")
_BIG_I32 = 0x7FFFFFFF


@functools.partial(
    pl.kernel,
    out_type=jax.ShapeDtypeStruct((NWORKERS, LANES), jnp.int32),
    mesh=plsc.VectorSubcoreMesh(core_axis_name="c", subcore_axis_name="s"),
    compiler_params=pltpu.CompilerParams(needs_layout_passes=False),
    scratch_types=[
        pltpu.VMEM((CHUNK,), jnp.float32),
        pltpu.VMEM((CHUNK,), jnp.float32),
        pltpu.VMEM((LANES,), jnp.int32),
        pltpu.SemaphoreType.DMA,
        pltpu.SemaphoreType.DMA,
    ],
)
def _sc_argmax(x_hbm, out_hbm, buf0, buf1, out_v, sem0, sem1):
    wid = lax.axis_index("s") * _NC + lax.axis_index("c")
    iota16 = lax.iota(jnp.int32, LANES)
    acc = jnp.zeros((LANES,), jnp.int32)
    bufs = (buf0, buf1)
    sems = (sem0, sem1)

    def start(g):
        row = wid * ROWS_PER_W + g // CHUNKS_PER_ROW
        off = row * COLS + (g % CHUNKS_PER_ROW) * CHUNK
        return pltpu.async_copy(
            x_hbm.at[pl.ds(off, CHUNK)], bufs[g % 2], sems[g % 2]
        )

    copies = {0: start(0), 1: start(1)}

    m = a = None
    for g in range(NCHUNKS):
        r, c = divmod(g, CHUNKS_PER_ROW)
        if c == 0:
            m = jnp.full((LANES,), _NEG_INF, jnp.float32)
            a = jnp.zeros((LANES,), jnp.int32)
        buf = bufs[g % 2]
        chunk_base = c * CHUNK
        copies[g].wait()

        def body(i, carry, buf=buf, chunk_base=chunk_base):
            mm, aa = carry
            ibase = i * (LANES * UNROLL)
            for u in range(UNROLL):
                off = ibase + u * LANES
                v = buf[pl.ds(off, LANES)]
                gt = v > mm
                mm = jnp.where(gt, v, mm)
                aa = jnp.where(gt, chunk_base + off, aa)
            return mm, aa

        m, a = lax.fori_loop(0, ITERS, body, (m, a))
        if g + 2 < NCHUNKS:
            copies[g + 2] = start(g + 2)

        if c == CHUNKS_PER_ROW - 1:
            best = jnp.max(m)
            cand = jnp.where(m == best, a + iota16, _BIG_I32)
            acc = jnp.where(iota16 == r, jnp.min(cand), acc)

    out_v[...] = acc
    pltpu.sync_copy(out_v, out_hbm.at[wid])


def kernel(m_logits):
    flat = m_logits.reshape(-1)
    out = _sc_argmax(flat)
    return out[:, :ROWS_PER_W].reshape(ROWS, 1)


# speedup vs baseline: 1.7524x; 1.0456x over previous
import functools

import jax
import jax.numpy as jnp
from jax import lax
from jax.experimental import pallas as pl
from jax.experimental.pallas import tpu as pltpu
from jax.experimental.pallas import tpu_sc as plsc

ROWS = 128
COLS = 100000
LANES = 16

_info = plsc.get_sparse_core_info()
_NC, _NS = _info.num_cores, _info.num_subcores
NWORKERS = _NC * _NS
ROWS_PER_W = ROWS // NWORKERS
CHUNK = 20000
CHUNKS_PER_ROW = COLS // CHUNK
NCHUNKS = ROWS_PER_W * CHUNKS_PER_ROW
UNROLL = 10
NACC = 5
ITERS = CHUNK // (LANES * UNROLL)

_NEG_INF = float("-inf")
_BIG_I32 = 0x7FFFFFFF


@functools.partial(
    pl.kernel,
    out_type=jax.ShapeDtypeStruct((NWORKERS, LANES), jnp.int32),
    mesh=plsc.VectorSubcoreMesh(core_axis_name="c", subcore_axis_name="s"),
    compiler_params=pltpu.CompilerParams(needs_layout_passes=False),
    scratch_types=[
        pltpu.VMEM((CHUNK,), jnp.float32),
        pltpu.VMEM((CHUNK,), jnp.float32),
        pltpu.VMEM((LANES,), jnp.int32),
        pltpu.SemaphoreType.DMA,
        pltpu.SemaphoreType.DMA,
    ],
)
def _sc_argmax(x_hbm, out_hbm, buf0, buf1, out_v, sem0, sem1):
    wid = lax.axis_index("s") * _NC + lax.axis_index("c")
    iota16 = lax.iota(jnp.int32, LANES)
    acc = jnp.zeros((LANES,), jnp.int32)
    bufs = (buf0, buf1)
    sems = (sem0, sem1)

    def start(g):
        row = wid * ROWS_PER_W + g // CHUNKS_PER_ROW
        off = row * COLS + (g % CHUNKS_PER_ROW) * CHUNK
        return pltpu.async_copy(
            x_hbm.at[pl.ds(off, CHUNK)], bufs[g % 2], sems[g % 2]
        )

    copies = {0: start(0), 1: start(1)}

    m = a = None
    for g in range(NCHUNKS):
        r, c = divmod(g, CHUNKS_PER_ROW)
        if c == 0:
            m = [jnp.full((LANES,), _NEG_INF, jnp.float32) for _ in range(NACC)]
            a = [jnp.zeros((LANES,), jnp.int32) for _ in range(NACC)]
        buf = bufs[g % 2]
        chunk_base = c * CHUNK
        copies[g].wait()

        def body(i, carry, buf=buf, chunk_base=chunk_base):
            mm, aa = list(carry[0]), list(carry[1])
            ibase = i * (LANES * UNROLL)
            for u in range(UNROLL):
                k = u % NACC
                off = ibase + u * LANES
                v = buf[pl.ds(off, LANES)]
                gt = v > mm[k]
                mm[k] = jnp.where(gt, v, mm[k])
                aa[k] = jnp.where(gt, chunk_base + off, aa[k])
            return tuple(mm), tuple(aa)

        m, a = lax.fori_loop(0, ITERS, body, (tuple(m), tuple(a)))
        m, a = list(m), list(a)
        if g + 2 < NCHUNKS:
            copies[g + 2] = start(g + 2)

        if c == CHUNKS_PER_ROW - 1:
            mm, aa = m[0], a[0] + iota16
            for k in range(1, NACC):
                ak = a[k] + iota16
                better = (m[k] > mm) | ((m[k] == mm) & (ak < aa))
                aa = jnp.where(better, ak, aa)
                mm = jnp.where(better, m[k], mm)
            best = jnp.max(mm)
            cand = jnp.where(mm == best, aa, _BIG_I32)
            acc = jnp.where(iota16 == r, jnp.min(cand), acc)

    out_v[...] = acc
    pltpu.sync_copy(out_v, out_hbm.at[wid])


def kernel(m_logits):
    flat = m_logits.reshape(-1)
    out = _sc_argmax(flat)
    return out[:, :ROWS_PER_W].reshape(ROWS, 1)


# speedup vs baseline: 5.9823x; 3.4138x over previous
import functools

import jax
import jax.numpy as jnp
from jax import lax
from jax.experimental import pallas as pl
from jax.experimental.pallas import tpu as pltpu
from jax.experimental.pallas import tpu_sc as plsc

ROWS = 128
COLS = 100000
LANES = 16

_info = plsc.get_sparse_core_info()
_NC, _NS = _info.num_cores, _info.num_subcores
NWORKERS = _NC * _NS
CW = 400
NBLOCKS = COLS // CW
FULL_J = (NBLOCKS // NWORKERS)
LAST_W = NBLOCKS - FULL_J * NWORKERS
NRB = ROWS // LANES

_NEG_INF = float("-inf")
_BIG_I32 = 0x7FFFFFFF


@functools.partial(
    pl.kernel,
    out_type=(
        jax.ShapeDtypeStruct((NWORKERS, ROWS), jnp.float32),
        jax.ShapeDtypeStruct((NWORKERS, ROWS), jnp.int32),
    ),
    mesh=plsc.VectorSubcoreMesh(core_axis_name="c", subcore_axis_name="s"),
    compiler_params=pltpu.CompilerParams(needs_layout_passes=False),
    scratch_types=[
        pltpu.VMEM((CW, ROWS), jnp.float32),
        pltpu.VMEM((CW, ROWS), jnp.float32),
        pltpu.VMEM((ROWS,), jnp.float32),
        pltpu.VMEM((ROWS,), jnp.int32),
        pltpu.VMEM_SHARED((_NS, ROWS), jnp.float32),
        pltpu.VMEM_SHARED((_NS, ROWS), jnp.int32),
        pltpu.SemaphoreType.DMA,
        pltpu.SemaphoreType.DMA,
    ],
)
def _sc_argmax(xt_hbm, oval_hbm, oidx_hbm, buf0, buf1, vbuf, ibuf,
               sh_val, sh_idx, sem0, sem1):
    cid = lax.axis_index("c")
    sid = lax.axis_index("s")
    wid = cid * _NS + sid
    bufs = (buf0, buf1)
    sems = (sem0, sem1)
    iota16 = lax.iota(jnp.int32, LANES)

    def start(j):
        blk = j * NWORKERS + wid
        off = pl.multiple_of(blk * CW, 8)
        return pltpu.async_copy(
            xt_hbm.at[pl.ds(off, CW), :], bufs[j % 2], sems[j % 2]
        )

    copies = {0: start(0), 1: start(1)}

    m = [jnp.full((LANES,), _NEG_INF, jnp.float32) for _ in range(NRB)]
    a = [jnp.zeros((LANES,), jnp.int32) for _ in range(NRB)]

    def block_pass(j, m, a):
        buf = bufs[j % 2]
        col0 = (j * NWORKERS + wid) * CW

        def body(cc, carry):
            mm = list(carry[0])
            aa = list(carry[1])
            col = col0 + cc
            for rb in range(NRB):
                x = buf[cc, pl.ds(rb * LANES, LANES)]
                gt = x > mm[rb]
                mm[rb] = jnp.where(gt, x, mm[rb])
                aa[rb] = jnp.where(gt, col, aa[rb])
            return tuple(mm), tuple(aa)

        m, a = lax.fori_loop(0, CW, body, (tuple(m), tuple(a)))
        return list(m), list(a)

    for j in range(FULL_J):
        copies[j].wait()
        m, a = block_pass(j, m, a)
        if j + 2 < FULL_J:
            copies[j + 2] = start(j + 2)
        elif j + 2 == FULL_J:
            @pl.when(wid < LAST_W)
            def _():
                start(FULL_J)

    copies[FULL_J % 2] = pltpu.make_async_copy(
        xt_hbm.at[pl.ds(0, CW), :], bufs[FULL_J % 2], sems[FULL_J % 2]
    )

    @pl.when(wid < LAST_W)
    def _():
        copies[FULL_J % 2].wait()

    buf = bufs[FULL_J % 2]
    col0 = jnp.where(wid < LAST_W, (FULL_J * NWORKERS + wid) * CW, 0)

    def tail_body(cc, carry):
        mm = list(carry[0])
        aa = list(carry[1])
        col = col0 + cc
        for rb in range(NRB):
            x = buf[cc, pl.ds(rb * LANES, LANES)]
            live = (wid < LAST_W) & (x > mm[rb])
            mm[rb] = jnp.where(live, x, mm[rb])
            aa[rb] = jnp.where(live, col, aa[rb])
        return tuple(mm), tuple(aa)

    m, a = lax.fori_loop(0, CW, tail_body, (tuple(m), tuple(a)))
    m, a = list(m), list(a)

    for rb in range(NRB):
        vbuf[pl.ds(rb * LANES, LANES)] = m[rb]
        ibuf[pl.ds(rb * LANES, LANES)] = a[rb]
    pltpu.sync_copy(vbuf, sh_val.at[sid])
    pltpu.sync_copy(ibuf, sh_idx.at[sid])
    plsc.subcore_barrier()

    @pl.when(sid == 0)
    def _():
        row0 = pl.multiple_of(cid * _NS, 8)
        pltpu.sync_copy(sh_val, oval_hbm.at[pl.ds(row0, _NS), :])
        pltpu.sync_copy(sh_idx, oidx_hbm.at[pl.ds(row0, _NS), :])


def _merge_body(val_ref, idx_ref, out_ref):
    v = val_ref[...]
    i = idx_ref[...]
    best = jnp.max(v, axis=0, keepdims=True)
    cand = jnp.where(v == best, i, _BIG_I32)
    out_ref[...] = jnp.min(cand, axis=0, keepdims=True)


_merge = pl.pallas_call(
    _merge_body,
    out_shape=jax.ShapeDtypeStruct((1, ROWS), jnp.int32),
)


def kernel(m_logits):
    xt = m_logits.T
    val, idx = _sc_argmax(xt)
    return _merge(val, idx).reshape(ROWS, 1)
